# baseline (speedup 1.0000x reference)
import dataclasses
import functools

import jax
import jax.numpy as jnp
import numpy as np
from jax import lax
from jax.experimental import pallas as pl
from jax.experimental.pallas import tpu as pltpu
from jax.experimental.pallas import tpu_sc as plsc

BATCH = 16384
N_NUM = 13
SEQ = 50
VOCAB = 100000
DIM = 16
MAX_NORM = 5.0
BN_EPS = 1e-5

NC = 2
NS = 16
NW = NC * NS
B = BATCH * SEQ
B_PER_W = B // NW
CHUNK = 256
N_CHUNKS = B_PER_W // CHUNK

ROWS_PER_LINE = 128 // DIM
LINES = VOCAB // ROWS_PER_LINE


def _bn_body(x_ref, g_ref, b_ref, o_ref):
    x = x_ref[...]
    mean = jnp.mean(x, axis=0, keepdims=True)
    xc = x - mean
    var = jnp.mean(xc * xc, axis=0, keepdims=True)
    inv = lax.rsqrt(var + BN_EPS)
    o_ref[...] = xc * inv * g_ref[...] + b_ref[...]


def _batchnorm(numeric, gamma, beta):
    return pl.pallas_call(
        _bn_body,
        out_shape=jax.ShapeDtypeStruct((BATCH, N_NUM), jnp.float32),
    )(numeric, gamma.reshape(1, N_NUM), beta.reshape(1, N_NUM))


_SEG = np.zeros((128, ROWS_PER_LINE), dtype=np.float32)
for _c in range(128):
    _SEG[_c, _c // DIM] = 1.0
_SEG_T = _SEG.T.copy()


def _scale_body(seg_ref, seg_t_ref, t_ref, o_ref):
    x = t_ref[...]
    n2 = lax.dot(x * x, seg_ref[...], preferred_element_type=jnp.float32)
    n = jnp.sqrt(n2)
    s8 = jnp.minimum(1.0, MAX_NORM / jnp.maximum(n, 1e-12))
    s = lax.dot(s8, seg_t_ref[...], preferred_element_type=jnp.float32)
    o_ref[...] = x * s


def _scale_tables(tables128):
    call = pl.pallas_call(
        _scale_body,
        out_shape=jax.ShapeDtypeStruct((LINES, 128), jnp.float32),
    )
    seg = jnp.asarray(_SEG)
    seg_t = jnp.asarray(_SEG_T)
    return [call(seg, seg_t, t) for t in tables128]


OUT_LINES = B * DIM // 128
LINES_PER_CHUNK = CHUNK * DIM // 128


def _gather4(tables128, indices):
    mesh = plsc.VectorSubcoreMesh(core_axis_name="c", subcore_axis_name="s")
    cp = pltpu.CompilerParams()
    if "needs_layout_passes" in pltpu.CompilerParams.__dataclass_fields__:
        cp = dataclasses.replace(cp, needs_layout_passes=False)

    @functools.partial(
        pl.kernel,
        mesh=mesh,
        compiler_params=cp,
        out_type=[jax.ShapeDtypeStruct((OUT_LINES, 128), jnp.float32)] * 4,
        scratch_types=[
            pltpu.VMEM((B_PER_W,), jnp.int32),
            pltpu.VMEM((CHUNK,), jnp.int32),
            pltpu.VMEM((CHUNK,), jnp.int32),
            pltpu.VMEM((CHUNK, 128), jnp.float32),
            pltpu.VMEM((CHUNK, 128), jnp.float32),
            pltpu.VMEM((LINES_PER_CHUNK, 128), jnp.float32),
            pltpu.SemaphoreType.DMA,
            pltpu.SemaphoreType.DMA,
        ],
    )
    def k(t0, t1, t2, t3, i0, i1, i2, i3, o0, o1, o2, o3,
          idxt_v, sup0_v, sup1_v, rows0_v, rows1_v, out_v, sem0, sem1):
        wid = lax.axis_index("s") * NC + lax.axis_index("c")
        base0 = wid * B_PER_W
        lane = lax.iota(jnp.int32, DIM)

        for t_hbm, i_hbm, o_hbm in ((t0, i0, o0), (t1, i1, o1),
                                    (t2, i2, o2), (t3, i3, o3)):
            pltpu.sync_copy(i_hbm.at[pl.ds(base0, B_PER_W)], idxt_v)

            def prep(c, sup_v, rows_v, sem, t_hbm=t_hbm):
                @pl.loop(0, CHUNK, step=DIM)
                def _(j):
                    sup_v[pl.ds(j, DIM)] = \
                        idxt_v[pl.ds(c * CHUNK + j, DIM)] >> 3
                pltpu.make_async_copy(t_hbm.at[sup_v], rows_v, sem).start()

            def drain(c, rows_v, sem, o_hbm=o_hbm, t_hbm=t_hbm):
                sup = sup0_v if rows_v is rows0_v else sup1_v
                pltpu.make_async_copy(t_hbm.at[sup], rows_v, sem).wait()

                @pl.loop(0, CHUNK, step=DIM)
                def _(j):
                    idxv = idxt_v[pl.ds(c * CHUNK + j, DIM)]
                    rowv = j + lane
                    colv = (idxv & 7) * DIM
                    addrv = rowv * DIM
                    for m in range(DIM):
                        vals = plsc.load_gather(rows_v, [rowv, colv + m])
                        a = addrv + m
                        plsc.store_scatter(out_v, [a >> 7, a & 127], vals)

                line0 = pl.multiple_of(
                    (base0 + c * CHUNK) * DIM // 128, 8)
                pltpu.sync_copy(
                    out_v, o_hbm.at[pl.ds(line0, LINES_PER_CHUNK), :])

            prep(0, sup0_v, rows0_v, sem0)

            @pl.loop(0, N_CHUNKS, step=2)
            def _(c, prep=prep, drain=drain):
                prep(c + 1, sup1_v, rows1_v, sem1)
                drain(c, rows0_v, sem0)

                @pl.when(c + 2 < N_CHUNKS)
                def _():
                    prep(c + 2, sup0_v, rows0_v, sem0)

                drain(c + 1, rows1_v, sem1)

    return k(*tables128, *indices)


def _to3d_body(x0, x1, x2, x3, o0, o1, o2, o3):
    for x_ref, o_ref in ((x0, o0), (x1, o1), (x2, o2), (x3, o3)):
        o_ref[...] = x_ref[...].reshape(o_ref.shape)


def _to_3d(flats):
    grid = 256
    bs = BATCH // grid
    in_spec = pl.BlockSpec((bs * SEQ, DIM), lambda i: (i, 0))
    out_spec = pl.BlockSpec((bs, SEQ, DIM), lambda i: (i, 0, 0))
    return pl.pallas_call(
        _to3d_body,
        grid=(grid,),
        in_specs=[in_spec] * 4,
        out_specs=[out_spec] * 4,
        out_shape=[jax.ShapeDtypeStruct((BATCH, SEQ, DIM), jnp.float32)] * 4,
    )(*[f.reshape(B, DIM) for f in flats])


def kernel(numeric, categorical, text_0, text_1, text_2, text_3,
           gamma, beta, table_0, table_1, table_2, table_3):
    numeric_out = _batchnorm(numeric, gamma, beta)
    t128 = [t.reshape(LINES, 128)
            for t in (table_0, table_1, table_2, table_3)]
    scaled = _scale_tables(t128)
    idx = [t.reshape(-1).astype(jnp.int32)
           for t in (text_0, text_1, text_2, text_3)]
    outs = _gather4(scaled, idx)
    embs = _to_3d(outs)
    return (numeric_out, categorical, embs[0], embs[1], embs[2], embs[3])


# speedup vs baseline: 1.7220x; 1.7220x over previous
import dataclasses
import functools

import jax
import jax.numpy as jnp
import numpy as np
from jax import lax
from jax.experimental import pallas as pl
from jax.experimental.pallas import tpu as pltpu
from jax.experimental.pallas import tpu_sc as plsc

BATCH = 16384
N_NUM = 13
SEQ = 50
VOCAB = 100000
DIM = 16
MAX_NORM = 5.0
BN_EPS = 1e-5

NC = 2
NS = 16
NW = NC * NS
B = BATCH * SEQ
B_PER_W = B // NW
CHUNK = 320
N_CHUNKS = B_PER_W // CHUNK

ROWS_PER_LINE = 128 // DIM
LINES = VOCAB // ROWS_PER_LINE


def _bn_body(x_ref, g_ref, b_ref, o_ref):
    x = x_ref[...]
    mean = jnp.mean(x, axis=0, keepdims=True)
    xc = x - mean
    var = jnp.mean(xc * xc, axis=0, keepdims=True)
    inv = lax.rsqrt(var + BN_EPS)
    o_ref[...] = xc * inv * g_ref[...] + b_ref[...]


def _batchnorm(numeric, gamma, beta):
    return pl.pallas_call(
        _bn_body,
        out_shape=jax.ShapeDtypeStruct((BATCH, N_NUM), jnp.float32),
    )(numeric, gamma.reshape(1, N_NUM), beta.reshape(1, N_NUM))


_SEG = np.zeros((128, ROWS_PER_LINE), dtype=np.float32)
for _c in range(128):
    _SEG[_c, _c // DIM] = 1.0
_SEG_T = _SEG.T.copy()


def _scale_body(seg_ref, seg_t_ref, t_ref, o_ref):
    x = t_ref[...]
    n2 = lax.dot(x * x, seg_ref[...], preferred_element_type=jnp.float32)
    n = jnp.sqrt(n2)
    s8 = jnp.minimum(1.0, MAX_NORM / jnp.maximum(n, 1e-12))
    s = lax.dot(s8, seg_t_ref[...], preferred_element_type=jnp.float32)
    o_ref[...] = x * s


def _scale_tables(tables128):
    call = pl.pallas_call(
        _scale_body,
        out_shape=jax.ShapeDtypeStruct((LINES, 128), jnp.float32),
    )
    seg = jnp.asarray(_SEG)
    seg_t = jnp.asarray(_SEG_T)
    return [call(seg, seg_t, t) for t in tables128]


OUT_LINES = B * DIM // 128
LINES_PER_CHUNK = CHUNK * DIM // 128


def _gather4(tables128, indices):
    mesh = plsc.VectorSubcoreMesh(core_axis_name="c", subcore_axis_name="s")
    cp = pltpu.CompilerParams()
    if "needs_layout_passes" in pltpu.CompilerParams.__dataclass_fields__:
        cp = dataclasses.replace(cp, needs_layout_passes=False)

    @functools.partial(
        pl.kernel,
        mesh=mesh,
        compiler_params=cp,
        out_type=[jax.ShapeDtypeStruct((OUT_LINES, 128), jnp.float32)] * 4,
        scratch_types=[
            pltpu.VMEM((B_PER_W,), jnp.int32),
            pltpu.VMEM((CHUNK,), jnp.int32),
            pltpu.VMEM((CHUNK,), jnp.int32),
            pltpu.VMEM((CHUNK, 128), jnp.float32),
            pltpu.VMEM((CHUNK, 128), jnp.float32),
            pltpu.VMEM((LINES_PER_CHUNK, 128), jnp.float32),
            pltpu.SemaphoreType.DMA,
            pltpu.SemaphoreType.DMA,
        ],
    )
    def k(t0, t1, t2, t3, i0, i1, i2, i3, o0, o1, o2, o3,
          idxt_v, sup0_v, sup1_v, rows0_v, rows1_v, out_v, sem0, sem1):
        wid = lax.axis_index("s") * NC + lax.axis_index("c")
        base0 = wid * B_PER_W
        lane = lax.iota(jnp.int32, DIM)

        for t_hbm, i_hbm, o_hbm in ((t0, i0, o0), (t1, i1, o1),
                                    (t2, i2, o2), (t3, i3, o3)):
            pltpu.sync_copy(i_hbm.at[pl.ds(base0, B_PER_W)], idxt_v)

            def prep(c, sup_v, rows_v, sem, t_hbm=t_hbm):
                @pl.loop(0, CHUNK, step=DIM)
                def _(j):
                    sup_v[pl.ds(j, DIM)] = \
                        idxt_v[pl.ds(c * CHUNK + j, DIM)] >> 3
                pltpu.make_async_copy(t_hbm.at[sup_v], rows_v, sem).start()

            def drain(c, rows_v, sem, o_hbm=o_hbm, t_hbm=t_hbm):
                sup = sup0_v if rows_v is rows0_v else sup1_v
                pltpu.make_async_copy(t_hbm.at[sup], rows_v, sem).wait()

                @pl.loop(0, CHUNK, step=DIM)
                def _(j):
                    idxv = idxt_v[pl.ds(c * CHUNK + j, DIM)]
                    rowv = j + lane
                    colv = (idxv & 7) * DIM
                    addrv = rowv * DIM
                    for m in range(DIM):
                        vals = plsc.load_gather(rows_v, [rowv, colv + m])
                        a = addrv + m
                        plsc.store_scatter(out_v, [a >> 7, a & 127], vals)

                line0 = pl.multiple_of(
                    (base0 + c * CHUNK) * DIM // 128, 8)
                pltpu.sync_copy(
                    out_v, o_hbm.at[pl.ds(line0, LINES_PER_CHUNK), :])

            prep(0, sup0_v, rows0_v, sem0)

            @pl.loop(0, N_CHUNKS, step=2)
            def _(c, prep=prep, drain=drain):
                prep(c + 1, sup1_v, rows1_v, sem1)
                drain(c, rows0_v, sem0)

                @pl.when(c + 2 < N_CHUNKS)
                def _():
                    prep(c + 2, sup0_v, rows0_v, sem0)

                drain(c + 1, rows1_v, sem1)

    return k(*tables128, *indices)


def _to3d_body(x0, x1, x2, x3, o0, o1, o2, o3):
    for x_ref, o_ref in ((x0, o0), (x1, o1), (x2, o2), (x3, o3)):
        o_ref[...] = x_ref[...].reshape(o_ref.shape)


def _to_3d(flats):
    grid = 256
    bs = BATCH // grid
    in_spec = pl.BlockSpec((bs * SEQ, DIM), lambda i: (i, 0))
    out_spec = pl.BlockSpec((bs, SEQ, DIM), lambda i: (i, 0, 0))
    return pl.pallas_call(
        _to3d_body,
        grid=(grid,),
        in_specs=[in_spec] * 4,
        out_specs=[out_spec] * 4,
        out_shape=[jax.ShapeDtypeStruct((BATCH, SEQ, DIM), jnp.float32)] * 4,
    )(*[f.reshape(B, DIM) for f in flats])


def kernel(numeric, categorical, text_0, text_1, text_2, text_3,
           gamma, beta, table_0, table_1, table_2, table_3):
    numeric_out = _batchnorm(numeric, gamma, beta)
    t128 = [t.reshape(LINES, 128)
            for t in (table_0, table_1, table_2, table_3)]
    scaled = _scale_tables(t128)
    idx = [t.reshape(-1).astype(jnp.int32)
           for t in (text_0, text_1, text_2, text_3)]
    outs = _gather4(scaled, idx)
    embs = [o.reshape(BATCH, SEQ, DIM) for o in outs]
    return (numeric_out, categorical, embs[0], embs[1], embs[2], embs[3])


# speedup vs baseline: 1.7235x; 1.0009x over previous
import dataclasses
import functools

import jax
import jax.numpy as jnp
import numpy as np
from jax import lax
from jax.experimental import pallas as pl
from jax.experimental.pallas import tpu as pltpu
from jax.experimental.pallas import tpu_sc as plsc

BATCH = 16384
N_NUM = 13
SEQ = 50
VOCAB = 100000
DIM = 16
MAX_NORM = 5.0
BN_EPS = 1e-5

NC = 2
NS = 16
NW = NC * NS
B = BATCH * SEQ
B_PER_W = B // NW
CHUNK = 320
N_CHUNKS = B_PER_W // CHUNK

ROWS_PER_LINE = 128 // DIM
LINES = VOCAB // ROWS_PER_LINE


def _bn_body(x_ref, g_ref, b_ref, o_ref):
    x = x_ref[...]
    mean = jnp.mean(x, axis=0, keepdims=True)
    xc = x - mean
    var = jnp.mean(xc * xc, axis=0, keepdims=True)
    inv = lax.rsqrt(var + BN_EPS)
    o_ref[...] = xc * inv * g_ref[...] + b_ref[...]


def _batchnorm(numeric, gamma, beta):
    return pl.pallas_call(
        _bn_body,
        out_shape=jax.ShapeDtypeStruct((BATCH, N_NUM), jnp.float32),
    )(numeric, gamma.reshape(1, N_NUM), beta.reshape(1, N_NUM))


_SEG = np.zeros((128, ROWS_PER_LINE), dtype=np.float32)
for _c in range(128):
    _SEG[_c, _c // DIM] = 1.0
_SEG_T = _SEG.T.copy()


def _scale_body(seg_ref, seg_t_ref, t_ref, o_ref):
    x = t_ref[...]
    n2 = lax.dot(x * x, seg_ref[...], preferred_element_type=jnp.float32)
    n = jnp.sqrt(n2)
    s8 = jnp.minimum(1.0, MAX_NORM / jnp.maximum(n, 1e-12))
    s = lax.dot(s8, seg_t_ref[...], preferred_element_type=jnp.float32)
    o_ref[...] = x * s


def _scale_tables(tables128):
    call = pl.pallas_call(
        _scale_body,
        out_shape=jax.ShapeDtypeStruct((LINES, 128), jnp.float32),
    )
    seg = jnp.asarray(_SEG)
    seg_t = jnp.asarray(_SEG_T)
    return [call(seg, seg_t, t) for t in tables128]


OUT_LINES = B * DIM // 128
LINES_PER_CHUNK = CHUNK * DIM // 128


def _gather4(tables128, indices):
    mesh = plsc.VectorSubcoreMesh(core_axis_name="c", subcore_axis_name="s")
    cp = pltpu.CompilerParams()
    if "needs_layout_passes" in pltpu.CompilerParams.__dataclass_fields__:
        cp = dataclasses.replace(cp, needs_layout_passes=False)

    @functools.partial(
        pl.kernel,
        mesh=mesh,
        compiler_params=cp,
        out_type=[jax.ShapeDtypeStruct((OUT_LINES, 128), jnp.float32)] * 4,
        scratch_types=[
            pltpu.VMEM((B_PER_W,), jnp.int32),
            pltpu.VMEM((CHUNK,), jnp.int32),
            pltpu.VMEM((CHUNK,), jnp.int32),
            pltpu.VMEM((CHUNK, 128), jnp.float32),
            pltpu.VMEM((CHUNK, 128), jnp.float32),
            pltpu.VMEM((LINES_PER_CHUNK, 128), jnp.float32),
            pltpu.SemaphoreType.DMA,
            pltpu.SemaphoreType.DMA,
        ],
    )
    def k(t0, t1, t2, t3, i0, i1, i2, i3, o0, o1, o2, o3,
          idxt_v, sup0_v, sup1_v, rows0_v, rows1_v, out_v, sem0, sem1):
        wid = lax.axis_index("s") * NC + lax.axis_index("c")
        base0 = wid * B_PER_W
        lane = lax.iota(jnp.int32, DIM)

        for t_hbm, i_hbm, o_hbm in ((t0, i0, o0), (t1, i1, o1),
                                    (t2, i2, o2), (t3, i3, o3)):
            pltpu.sync_copy(i_hbm.at[pl.ds(base0, B_PER_W)], idxt_v)

            def prep(c, sup_v, rows_v, sem, t_hbm=t_hbm):
                @pl.loop(0, CHUNK, step=DIM)
                def _(j):
                    sup_v[pl.ds(j, DIM)] = \
                        idxt_v[pl.ds(c * CHUNK + j, DIM)] >> 3
                pltpu.make_async_copy(t_hbm.at[sup_v], rows_v, sem).start()

            def drain(c, rows_v, sem, o_hbm=o_hbm, t_hbm=t_hbm):
                sup = sup0_v if rows_v is rows0_v else sup1_v
                pltpu.make_async_copy(t_hbm.at[sup], rows_v, sem).wait()

                @pl.loop(0, CHUNK, step=DIM)
                def _(j):
                    idxv = idxt_v[pl.ds(c * CHUNK + j, DIM)]
                    rowv = j + lane
                    colv = (idxv & 7) * DIM
                    addrv = rowv * DIM
                    for m in range(DIM):
                        vals = plsc.load_gather(rows_v, [rowv, colv + m])
                        a = addrv + m
                        plsc.store_scatter(out_v, [a >> 7, a & 127], vals)

                line0 = pl.multiple_of(
                    (base0 + c * CHUNK) * DIM // 128, 8)
                pltpu.sync_copy(
                    out_v, o_hbm.at[pl.ds(line0, LINES_PER_CHUNK), :])

            prep(0, sup0_v, rows0_v, sem0)

            @pl.loop(0, N_CHUNKS, step=2)
            def _(c, prep=prep, drain=drain):
                prep(c + 1, sup1_v, rows1_v, sem1)
                drain(c, rows0_v, sem0)

                @pl.when(c + 2 < N_CHUNKS)
                def _():
                    prep(c + 2, sup0_v, rows0_v, sem0)

                drain(c + 1, rows1_v, sem1)

    return k(*tables128, *indices)


def kernel(numeric, categorical, text_0, text_1, text_2, text_3,
           gamma, beta, table_0, table_1, table_2, table_3):
    numeric_out = _batchnorm(numeric, gamma, beta)
    t128 = [t.reshape(LINES, 128)
            for t in (table_0, table_1, table_2, table_3)]
    scaled = _scale_tables(t128)
    idx = [t.reshape(-1).astype(jnp.int32)
           for t in (text_0, text_1, text_2, text_3)]
    outs = _gather4(scaled, idx)
    embs = [o.reshape(BATCH, SEQ, DIM) for o in outs]
    return (numeric_out, categorical, embs[0], embs[1], embs[2], embs[3])
